# Initial kernel scaffold; baseline (speedup 1.0000x reference)
import jax
import jax.numpy as jnp
from jax.experimental import pallas as pl


def kernel(hidden_states, gate_w, gate_up_proj, down_proj):
    raise NotImplementedError("write your pallas kernel here")


# speedup vs baseline: 1.1019x; 1.1019x over previous
import jax
import jax.numpy as jnp
from jax.experimental import pallas as pl
from jax.experimental.pallas import tpu as pltpu

_B, _S, _H, _E, _F = 1, 2048, 2048, 8, 768
_TT = 512


def _moe_body(hs_ref, gw_ref, w1_ref, w2_ref, out_ref, logits_ref):
    e = pl.program_id(1)
    x = hs_ref[...]
    gw = gw_ref[...]
    logits = jax.lax.dot_general(
        x, gw, (((1,), (1,)), ((), ())),
        preferred_element_type=jnp.float32)

    @pl.when(e == 0)
    def _():
        logits_ref[...] = logits

    p = jax.nn.softmax(logits, axis=-1)
    eio = jax.lax.broadcasted_iota(jnp.int32, p.shape, 1)
    m1 = jnp.max(p, axis=-1, keepdims=True)
    i1 = jnp.argmax(p, axis=-1)[:, None]
    oh1 = eio == i1
    p2 = jnp.where(oh1, -jnp.inf, p)
    m2 = jnp.max(p2, axis=-1, keepdims=True)
    i2 = jnp.argmax(p2, axis=-1)[:, None]
    oh2 = eio == i2
    wdense = (jnp.where(oh1, m1, 0.0) + jnp.where(oh2, m2, 0.0)) / (m1 + m2)
    we = jnp.sum(jnp.where(eio == e, wdense, 0.0), axis=-1, keepdims=True)

    gu = jnp.dot(x, w1_ref[0], preferred_element_type=jnp.float32)
    g = gu[:, :_F]
    u = gu[:, _F:]
    inter = u * (g * jax.nn.sigmoid(g))
    y = jnp.dot(inter, w2_ref[0], preferred_element_type=jnp.float32)
    contrib = we * y

    @pl.when(e == 0)
    def _():
        out_ref[...] = contrib

    @pl.when(e > 0)
    def _():
        out_ref[...] += contrib


def kernel(hidden_states, gate_w, gate_up_proj, down_proj):
    T = _B * _S
    hs = hidden_states.reshape(T, _H)
    out, logits = pl.pallas_call(
        _moe_body,
        grid=(T // _TT, _E),
        in_specs=[
            pl.BlockSpec((_TT, _H), lambda t, e: (t, 0)),
            pl.BlockSpec((_E, _H), lambda t, e: (0, 0)),
            pl.BlockSpec((1, _H, 2 * _F), lambda t, e: (e, 0, 0)),
            pl.BlockSpec((1, _F, _H), lambda t, e: (e, 0, 0)),
        ],
        out_specs=[
            pl.BlockSpec((_TT, _H), lambda t, e: (t, 0)),
            pl.BlockSpec((_TT, _E), lambda t, e: (t, 0)),
        ],
        out_shape=[
            jax.ShapeDtypeStruct((T, _H), jnp.float32),
            jax.ShapeDtypeStruct((T, _E), jnp.float32),
        ],
        compiler_params=pltpu.CompilerParams(
            dimension_semantics=("arbitrary", "arbitrary")),
    )(hs, gate_w, gate_up_proj, down_proj)
    return out.reshape(_B, _S, _H), logits
